# Initial kernel scaffold; baseline (speedup 1.0000x reference)
import jax
import jax.numpy as jnp
from jax.experimental import pallas as pl


def kernel(raw_x, w_real, w_imag, conv1_w, conv1_b, conv2_w, conv2_b, msg_w1, msg_b1, msg_w2, msg_b2, loc_w, loc_b, s2f_w, s2f_b, gin_w, gin_b, gru_wih, gru_whh, gru_bih, gru_bhh, cls_w, cls_b):
    raise NotImplementedError("write your pallas kernel here")


# speedup vs baseline: 21.7115x; 21.7115x over previous
import functools
import math

import jax
import jax.numpy as jnp
import numpy as np
from jax.experimental import pallas as pl
from jax.experimental.pallas import tpu as pltpu

C = 48
F = 16
M = 3
H = 32
ENC = 16
STRIDE = 10
THETA = math.radians(45.0)
N_CLASSES = 4
K1 = 5
WIN = 9
CF = C * F


def _step_kernel(
    wr_ref,
    wi_ref,
    rawt_ref,
    rawwin_ref,
    mask_ref,
    p_ref,
    d_ref,
    w1x_ref,
    w1s_ref,
    w1d_ref,
    w1r_ref,
    b1_ref,
    w2_ref,
    b2_ref,
    s2f_ref,
    s2fb_ref,
    w1cat_ref,
    b1cat_ref,
    w2cat_ref,
    cb2_ref,
    ginp_ref,
    locgin_ref,
    blg_ref,
    wihT_ref,
    whhT_ref,
    bih_ref,
    bhh_ref,
    cls_w_ref,
    cls_b_ref,
    logits_ref,
    gates_ref,
    state_ref,
    psum_ref,
    gsum_ref,
    *, n_steps):
    t = pl.program_id(1)

    @pl.when(t == 0)
    def _init():
        state_ref[...] = jnp.zeros_like(state_ref)
        psum_ref[...] = jnp.zeros_like(psum_ref)
        gsum_ref[...] = jnp.zeros_like(gsum_ref)

    wr = wr_ref[0, 0]
    wi = wi_ref[0, 0]
    p = p_ref[...]
    dmask = d_ref[...]

    def expand(x):
        return jnp.dot(x, p, preferred_element_type=jnp.float32)

    def src_row(x_exp):
        return jnp.sum(x_exp * dmask, axis=0, keepdims=True)

    wr_d = expand(wr)
    wi_d = expand(wi)
    wr_srow = src_row(wr_d)
    wi_srow = src_row(wi_d)

    xr = wr_srow * wr_d + wi_srow * wi_d
    xi = wi_srow * wr_d - wr_srow * wi_d

    xmag = jnp.sqrt(xr * xr + xi * xi + 1e-12)
    xlog = jnp.log1p(xmag)
    gate = jnp.where((xi > 0.0) & ((xr <= 0.0) | (xi > xr)), 1.0, 0.0)
    gsum_ref[...] += jnp.sum(gate, keepdims=True)

    state = state_ref[...]
    sf = jnp.dot(state, s2f_ref[...],
                 preferred_element_type=jnp.float32) + s2fb_ref[...]
    sf_m = [sf[:, j * F:(j + 1) * F] for j in range(M)]
    rawt = rawt_ref[0, 0]

    w1s = w1s_ref[...]
    w1d = w1d_ref[...]
    w1r = w1r_ref[...]
    b1 = b1_ref[...]
    w1x = w1x_ref[...]
    w2 = w2_ref[...]
    b2 = b2_ref[...]

    msg2 = [None] * M
    a_s = [None] * M
    a_d = [None] * M
    for m in range(M):
        asm = sf_m[0] * w1s[0, m]
        adm = sf_m[0] * w1d[0, m]
        for j in range(1, M):
            asm = asm + sf_m[j] * w1s[j, m]
            adm = adm + sf_m[j] * w1d[j, m]
        asm = asm + rawt * w1r[0, m]
        adm = adm + rawt * w1r[1, m] + b1[0, m]
        a_s[m] = asm
        a_d[m] = adm

    h1 = [None] * M
    for m in range(M):
        a_s_row = src_row(expand(a_s[m]))
        a_d_exp = expand(a_d[m])
        pre = xlog * w1x[0, m] + a_s_row + a_d_exp
        h1[m] = jnp.maximum(pre, 0.0)

    pt = p.T
    agg = [None] * M
    for mo in range(M):
        acc = h1[0] * w2[0, mo]
        for mi in range(1, M):
            acc = acc + h1[mi] * w2[mi, mo]
        acc = (acc + b2[0, mo]) * gate
        agg[mo] = jnp.dot(acc, pt, preferred_element_type=jnp.float32)

    agg_cat = jnp.concatenate(agg, axis=1)

    win = rawwin_ref[0, 0]
    e1 = jnp.maximum(
        jnp.dot(win, w1cat_ref[...], preferred_element_type=jnp.float32)
        + b1cat_ref[...], 0.0) * mask_ref[0]
    enc_t = jnp.maximum(
        jnp.dot(e1, w2cat_ref[...], preferred_element_type=jnp.float32)
        + cb2_ref[...], 0.0)

    gru_in = (jnp.dot(agg_cat, ginp_ref[...], preferred_element_type=jnp.float32)
              + jnp.dot(enc_t, locgin_ref[...], preferred_element_type=jnp.float32)
              + blg_ref[...])

    gx = jnp.dot(gru_in, wihT_ref[...], preferred_element_type=jnp.float32) + bih_ref[...]
    gh = jnp.dot(state, whhT_ref[...], preferred_element_type=jnp.float32) + bhh_ref[...]
    xr_, xz_, xn_ = gx[:, :H], gx[:, H:2 * H], gx[:, 2 * H:]
    hr_, hz_, hn_ = gh[:, :H], gh[:, H:2 * H], gh[:, 2 * H:]
    r = jax.nn.sigmoid(xr_ + hr_)
    z = jax.nn.sigmoid(xz_ + hz_)
    n = jnp.tanh(xn_ + r * hn_)
    new_state = (1.0 - z) * n + z * state
    state_ref[...] = new_state

    pooled = jnp.sum(new_state, axis=0, keepdims=True) * (1.0 / C)
    psum_ref[...] += pooled

    @pl.when(t == n_steps - 1)
    def _finish():
        prev_mean = psum_ref[...] * (1.0 / n_steps)
        readout = jnp.concatenate([pooled, prev_mean], axis=1)
        logits_ref[0] = (
            jnp.dot(readout, cls_w_ref[...], preferred_element_type=jnp.float32)
            + cls_b_ref[...])
        gates_ref[0] = gsum_ref[...]


def kernel(raw_x, w_real, w_imag, conv1_w, conv1_b, conv2_w, conv2_b,
           msg_w1, msg_b1, msg_w2, msg_b2, loc_w, loc_b, s2f_w, s2f_b,
           gin_w, gin_b, gru_wih, gru_whh, gru_bih, gru_bhh, cls_w, cls_b):
    b, c, t_len = raw_x.shape
    n_steps = (t_len + STRIDE - 1) // STRIDE
    ts = np.arange(0, t_len, STRIDE)

    wr_s = jnp.transpose(w_real[:, :, ::STRIDE, :], (0, 2, 1, 3))
    wi_s = jnp.transpose(w_imag[:, :, ::STRIDE, :], (0, 2, 1, 3))
    raw_t = jnp.transpose(raw_x[:, :, ::STRIDE], (0, 2, 1))[..., None]

    pad = (WIN - 1) // 2
    raw_pad = jnp.pad(raw_x, ((0, 0), (0, 0), (pad, pad)))
    win_idx = ts[:, None] + np.arange(WIN)[None, :]
    raw_win = jnp.transpose(raw_pad[:, :, win_idx], (0, 2, 1, 3))

    w1cat = jnp.zeros((WIN, K1 * ENC), dtype=jnp.float32)
    for j in range(K1):
        w1cat = w1cat.at[j:j + K1, j * ENC:(j + 1) * ENC].set(conv1_w[:, 0, :].T)
    b1cat = jnp.tile(conv1_b, (K1,)).reshape(1, K1 * ENC)
    w2cat = jnp.transpose(conv2_w, (2, 1, 0)).reshape(K1 * ENC, ENC)

    mask_np = np.ones((n_steps, 1, K1 * ENC), dtype=np.float32)
    for si, t0 in enumerate(ts):
        for j in range(K1):
            if not (0 <= t0 + j - 2 < t_len):
                mask_np[si, 0, j * ENC:(j + 1) * ENC] = 0.0
    mask = jnp.asarray(mask_np)

    p_np = np.zeros((F, CF), dtype=np.float32)
    for s in range(C):
        p_np[:, s * F:(s + 1) * F] = np.eye(F, dtype=np.float32)
    p_sel = jnp.asarray(p_np)

    d_np = np.zeros((C, CF), dtype=np.float32)
    for s in range(C):
        d_np[s, s * F:(s + 1) * F] = 1.0
    d_sel = jnp.asarray(d_np)

    mi = 1 + 2 * M + 2
    w1x = msg_w1[0:1, :]
    w1s = msg_w1[1:1 + M, :]
    w1d = msg_w1[1 + M:1 + 2 * M, :]
    w1r = msg_w1[1 + 2 * M:mi, :]
    b1 = msg_b1.reshape(1, M)
    b2 = msg_b2.reshape(1, M)

    perm = np.arange(F * M).reshape(F, M).T.reshape(-1)
    s2fp = s2f_w[:, perm]
    s2fb = s2f_b[perm].reshape(1, F * M)
    ginp = gin_w[perm, :]
    locgin = loc_w @ gin_w
    blg = (loc_b @ gin_w + gin_b).reshape(1, H)
    wihT = gru_wih.T
    whhT = gru_whh.T
    bih = gru_bih.reshape(1, 3 * H)
    bhh = gru_bhh.reshape(1, 3 * H)
    cls_b2 = cls_b.reshape(1, N_CLASSES)

    def bspec(block, imap):
        return pl.BlockSpec(block, imap)

    grid = (b, n_steps)
    full = lambda *shape: pl.BlockSpec(shape, lambda bi, ti: (0,) * len(shape))

    out_shapes = (
        jax.ShapeDtypeStruct((b, 1, N_CLASSES), jnp.float32),
        jax.ShapeDtypeStruct((b, 1, 1), jnp.float32),
    )

    logits, gate_parts = pl.pallas_call(
        functools.partial(_step_kernel, n_steps=n_steps),
        grid=grid,
        in_specs=[
            bspec((1, 1, C, F), lambda bi, ti: (bi, ti, 0, 0)),
            bspec((1, 1, C, F), lambda bi, ti: (bi, ti, 0, 0)),
            bspec((1, 1, C, 1), lambda bi, ti: (bi, ti, 0, 0)),
            bspec((1, 1, C, WIN), lambda bi, ti: (bi, ti, 0, 0)),
            bspec((1, 1, K1 * ENC), lambda bi, ti: (ti, 0, 0)),
            full(F, CF),
            full(C, CF),
            full(1, M),
            full(M, M),
            full(M, M),
            full(2, M),
            full(1, M),
            full(M, M),
            full(1, M),
            full(H, F * M),
            full(1, F * M),
            full(WIN, K1 * ENC),
            full(1, K1 * ENC),
            full(K1 * ENC, ENC),
            full(1, ENC),
            full(F * M, H),
            full(ENC, H),
            full(1, H),
            full(H, 3 * H),
            full(H, 3 * H),
            full(1, 3 * H),
            full(1, 3 * H),
            full(2 * H, N_CLASSES),
            full(1, N_CLASSES),
        ],
        out_specs=[
            bspec((1, 1, N_CLASSES), lambda bi, ti: (bi, 0, 0)),
            bspec((1, 1, 1), lambda bi, ti: (bi, 0, 0)),
        ],
        out_shape=out_shapes,
        scratch_shapes=[
            pltpu.VMEM((C, H), jnp.float32),
            pltpu.VMEM((1, H), jnp.float32),
            pltpu.VMEM((1, 1), jnp.float32),
        ],
        compiler_params=pltpu.CompilerParams(
            dimension_semantics=("arbitrary", "arbitrary"),
        ),
    )(
        wr_s, wi_s, raw_t, raw_win, mask, p_sel, d_sel,
        w1x, w1s, w1d, w1r, b1, msg_w2, b2,
        s2fp, s2fb, w1cat, b1cat, w2cat, conv2_b.reshape(1, ENC),
        ginp, locgin, blg, wihT, whhT, bih, bhh, cls_w, cls_b2,
    )

    e_num = C * (C - 1)
    gate_count = float(n_steps) * float(b * e_num * F)
    gate_rate = jnp.sum(gate_parts) / gate_count
    return logits.reshape(b, N_CLASSES), gate_rate


# speedup vs baseline: 22.0156x; 1.0140x over previous
import functools
import math

import jax
import jax.numpy as jnp
import numpy as np
from jax.experimental import pallas as pl
from jax.experimental.pallas import tpu as pltpu

C = 48
F = 16
M = 3
H = 32
ENC = 16
STRIDE = 10
THETA = math.radians(45.0)
N_CLASSES = 4
K1 = 5
WIN = 9
CF = C * F


def _step_kernel(
    wr_ref,
    wi_ref,
    wrrow_ref,
    wirow_ref,
    rawt_ref,
    rawwin_ref,
    mask_ref,
    p_ref,
    d_ref,
    w1x_ref,
    w1s_ref,
    w1d_ref,
    w1r_ref,
    b1_ref,
    w2_ref,
    b2_ref,
    s2f_ref,
    s2fb_ref,
    w1cat_ref,
    b1cat_ref,
    w2cat_ref,
    cb2_ref,
    ginp_ref,
    locgin_ref,
    blg_ref,
    wihT_ref,
    whhT_ref,
    bih_ref,
    bhh_ref,
    cls_w_ref,
    cls_b_ref,
    logits_ref,
    gates_ref,
    state_ref,
    psum_ref,
    gsum_ref,
    *, n_steps):
    t = pl.program_id(1)

    @pl.when(t == 0)
    def _init():
        state_ref[...] = jnp.zeros_like(state_ref)
        psum_ref[...] = jnp.zeros_like(psum_ref)
        gsum_ref[...] = jnp.zeros_like(gsum_ref)

    wr = wr_ref[0, 0]
    wi = wi_ref[0, 0]
    p = p_ref[...]
    dmask = d_ref[...]

    def expand(x):
        return jnp.dot(x, p, preferred_element_type=jnp.float32)

    def src_row(x_exp):
        return jnp.sum(x_exp * dmask, axis=0, keepdims=True)

    wr_d = expand(wr)
    wi_d = expand(wi)
    wr_srow = wrrow_ref[0, 0]
    wi_srow = wirow_ref[0, 0]

    xr = wr_srow * wr_d + wi_srow * wi_d
    xi = wi_srow * wr_d - wr_srow * wi_d

    xmag = jnp.sqrt(xr * xr + xi * xi + 1e-12)
    xlog = jnp.log1p(xmag)
    gate = jnp.where((xi > 0.0) & ((xr <= 0.0) | (xi > xr)), 1.0, 0.0)
    gsum_ref[...] += jnp.sum(gate, keepdims=True)

    state = state_ref[...]
    sf = jnp.dot(state, s2f_ref[...],
                 preferred_element_type=jnp.float32) + s2fb_ref[...]
    sf_m = [sf[:, j * F:(j + 1) * F] for j in range(M)]
    rawt = rawt_ref[0, 0]

    w1s = w1s_ref[...]
    w1d = w1d_ref[...]
    w1r = w1r_ref[...]
    b1 = b1_ref[...]
    w1x = w1x_ref[...]
    w2 = w2_ref[...]
    b2 = b2_ref[...]

    msg2 = [None] * M
    a_s = [None] * M
    a_d = [None] * M
    for m in range(M):
        asm = sf_m[0] * w1s[0, m]
        adm = sf_m[0] * w1d[0, m]
        for j in range(1, M):
            asm = asm + sf_m[j] * w1s[j, m]
            adm = adm + sf_m[j] * w1d[j, m]
        asm = asm + rawt * w1r[0, m]
        adm = adm + rawt * w1r[1, m] + b1[0, m]
        a_s[m] = asm
        a_d[m] = adm

    h1 = [None] * M
    for m in range(M):
        a_s_row = src_row(expand(a_s[m]))
        a_d_exp = expand(a_d[m])
        pre = xlog * w1x[0, m] + a_s_row + a_d_exp
        h1[m] = jnp.maximum(pre, 0.0)

    pt = p.T
    agg = [None] * M
    for mo in range(M):
        acc = h1[0] * w2[0, mo]
        for mi in range(1, M):
            acc = acc + h1[mi] * w2[mi, mo]
        acc = (acc + b2[0, mo]) * gate
        agg[mo] = jnp.dot(acc, pt, preferred_element_type=jnp.float32)

    agg_cat = jnp.concatenate(agg, axis=1)

    win = rawwin_ref[0, 0]
    e1 = jnp.maximum(
        jnp.dot(win, w1cat_ref[...], preferred_element_type=jnp.float32)
        + b1cat_ref[...], 0.0) * mask_ref[0]
    enc_t = jnp.maximum(
        jnp.dot(e1, w2cat_ref[...], preferred_element_type=jnp.float32)
        + cb2_ref[...], 0.0)

    gru_in = (jnp.dot(agg_cat, ginp_ref[...], preferred_element_type=jnp.float32)
              + jnp.dot(enc_t, locgin_ref[...], preferred_element_type=jnp.float32)
              + blg_ref[...])

    gx = jnp.dot(gru_in, wihT_ref[...], preferred_element_type=jnp.float32) + bih_ref[...]
    gh = jnp.dot(state, whhT_ref[...], preferred_element_type=jnp.float32) + bhh_ref[...]
    xr_, xz_, xn_ = gx[:, :H], gx[:, H:2 * H], gx[:, 2 * H:]
    hr_, hz_, hn_ = gh[:, :H], gh[:, H:2 * H], gh[:, 2 * H:]
    r = jax.nn.sigmoid(xr_ + hr_)
    z = jax.nn.sigmoid(xz_ + hz_)
    n = jnp.tanh(xn_ + r * hn_)
    new_state = (1.0 - z) * n + z * state
    state_ref[...] = new_state

    pooled = jnp.sum(new_state, axis=0, keepdims=True) * (1.0 / C)
    psum_ref[...] += pooled

    @pl.when(t == n_steps - 1)
    def _finish():
        prev_mean = psum_ref[...] * (1.0 / n_steps)
        readout = jnp.concatenate([pooled, prev_mean], axis=1)
        logits_ref[0] = (
            jnp.dot(readout, cls_w_ref[...], preferred_element_type=jnp.float32)
            + cls_b_ref[...])
        gates_ref[0] = gsum_ref[...]


def kernel(raw_x, w_real, w_imag, conv1_w, conv1_b, conv2_w, conv2_b,
           msg_w1, msg_b1, msg_w2, msg_b2, loc_w, loc_b, s2f_w, s2f_b,
           gin_w, gin_b, gru_wih, gru_whh, gru_bih, gru_bhh, cls_w, cls_b):
    b, c, t_len = raw_x.shape
    n_steps = (t_len + STRIDE - 1) // STRIDE
    ts = np.arange(0, t_len, STRIDE)

    wr_s = jnp.transpose(w_real[:, :, ::STRIDE, :], (0, 2, 1, 3))
    wi_s = jnp.transpose(w_imag[:, :, ::STRIDE, :], (0, 2, 1, 3))
    wr_row = wr_s.reshape(b, n_steps, 1, C * F)
    wi_row = wi_s.reshape(b, n_steps, 1, C * F)
    raw_t = jnp.transpose(raw_x[:, :, ::STRIDE], (0, 2, 1))[..., None]

    pad = (WIN - 1) // 2
    raw_pad = jnp.pad(raw_x, ((0, 0), (0, 0), (pad, pad)))
    win_idx = ts[:, None] + np.arange(WIN)[None, :]
    raw_win = jnp.transpose(raw_pad[:, :, win_idx], (0, 2, 1, 3))

    w1cat = jnp.zeros((WIN, K1 * ENC), dtype=jnp.float32)
    for j in range(K1):
        w1cat = w1cat.at[j:j + K1, j * ENC:(j + 1) * ENC].set(conv1_w[:, 0, :].T)
    b1cat = jnp.tile(conv1_b, (K1,)).reshape(1, K1 * ENC)
    w2cat = jnp.transpose(conv2_w, (2, 1, 0)).reshape(K1 * ENC, ENC)

    mask_np = np.ones((n_steps, 1, K1 * ENC), dtype=np.float32)
    for si, t0 in enumerate(ts):
        for j in range(K1):
            if not (0 <= t0 + j - 2 < t_len):
                mask_np[si, 0, j * ENC:(j + 1) * ENC] = 0.0
    mask = jnp.asarray(mask_np)

    p_np = np.zeros((F, CF), dtype=np.float32)
    for s in range(C):
        p_np[:, s * F:(s + 1) * F] = np.eye(F, dtype=np.float32)
    p_sel = jnp.asarray(p_np)

    d_np = np.zeros((C, CF), dtype=np.float32)
    for s in range(C):
        d_np[s, s * F:(s + 1) * F] = 1.0
    d_sel = jnp.asarray(d_np)

    mi = 1 + 2 * M + 2
    w1x = msg_w1[0:1, :]
    w1s = msg_w1[1:1 + M, :]
    w1d = msg_w1[1 + M:1 + 2 * M, :]
    w1r = msg_w1[1 + 2 * M:mi, :]
    b1 = msg_b1.reshape(1, M)
    b2 = msg_b2.reshape(1, M)

    perm = np.arange(F * M).reshape(F, M).T.reshape(-1)
    s2fp = s2f_w[:, perm]
    s2fb = s2f_b[perm].reshape(1, F * M)
    ginp = gin_w[perm, :]
    locgin = loc_w @ gin_w
    blg = (loc_b @ gin_w + gin_b).reshape(1, H)
    wihT = gru_wih.T
    whhT = gru_whh.T
    bih = gru_bih.reshape(1, 3 * H)
    bhh = gru_bhh.reshape(1, 3 * H)
    cls_b2 = cls_b.reshape(1, N_CLASSES)

    def bspec(block, imap):
        return pl.BlockSpec(block, imap)

    grid = (b, n_steps)
    full = lambda *shape: pl.BlockSpec(shape, lambda bi, ti: (0,) * len(shape))

    out_shapes = (
        jax.ShapeDtypeStruct((b, 1, N_CLASSES), jnp.float32),
        jax.ShapeDtypeStruct((b, 1, 1), jnp.float32),
    )

    logits, gate_parts = pl.pallas_call(
        functools.partial(_step_kernel, n_steps=n_steps),
        grid=grid,
        in_specs=[
            bspec((1, 1, C, F), lambda bi, ti: (bi, ti, 0, 0)),
            bspec((1, 1, C, F), lambda bi, ti: (bi, ti, 0, 0)),
            bspec((1, 1, 1, CF), lambda bi, ti: (bi, ti, 0, 0)),
            bspec((1, 1, 1, CF), lambda bi, ti: (bi, ti, 0, 0)),
            bspec((1, 1, C, 1), lambda bi, ti: (bi, ti, 0, 0)),
            bspec((1, 1, C, WIN), lambda bi, ti: (bi, ti, 0, 0)),
            bspec((1, 1, K1 * ENC), lambda bi, ti: (ti, 0, 0)),
            full(F, CF),
            full(C, CF),
            full(1, M),
            full(M, M),
            full(M, M),
            full(2, M),
            full(1, M),
            full(M, M),
            full(1, M),
            full(H, F * M),
            full(1, F * M),
            full(WIN, K1 * ENC),
            full(1, K1 * ENC),
            full(K1 * ENC, ENC),
            full(1, ENC),
            full(F * M, H),
            full(ENC, H),
            full(1, H),
            full(H, 3 * H),
            full(H, 3 * H),
            full(1, 3 * H),
            full(1, 3 * H),
            full(2 * H, N_CLASSES),
            full(1, N_CLASSES),
        ],
        out_specs=[
            bspec((1, 1, N_CLASSES), lambda bi, ti: (bi, 0, 0)),
            bspec((1, 1, 1), lambda bi, ti: (bi, 0, 0)),
        ],
        out_shape=out_shapes,
        scratch_shapes=[
            pltpu.VMEM((C, H), jnp.float32),
            pltpu.VMEM((1, H), jnp.float32),
            pltpu.VMEM((1, 1), jnp.float32),
        ],
        compiler_params=pltpu.CompilerParams(
            dimension_semantics=("parallel", "arbitrary"),
        ),
    )(
        wr_s, wi_s, wr_row, wi_row, raw_t, raw_win, mask, p_sel, d_sel,
        w1x, w1s, w1d, w1r, b1, msg_w2, b2,
        s2fp, s2fb, w1cat, b1cat, w2cat, conv2_b.reshape(1, ENC),
        ginp, locgin, blg, wihT, whhT, bih, bhh, cls_w, cls_b2,
    )

    e_num = C * (C - 1)
    gate_count = float(n_steps) * float(b * e_num * F)
    gate_rate = jnp.sum(gate_parts) / gate_count
    return logits.reshape(b, N_CLASSES), gate_rate


# speedup vs baseline: 32.4480x; 1.4739x over previous
import functools
import math

import jax
import jax.numpy as jnp
import numpy as np
from jax.experimental import pallas as pl
from jax.experimental.pallas import tpu as pltpu

C = 48
F = 16
M = 3
H = 32
ENC = 16
STRIDE = 10
THETA = math.radians(45.0)
N_CLASSES = 4
K1 = 5
WIN = 9
CF = C * F
G = 4
GC = G * C


def _step_kernel(
    wr_ref,
    wi_ref,
    wrrow_ref,
    wirow_ref,
    rawt_ref,
    rawwin_ref,
    mask_ref,
    p_ref,
    dmt_ref,
    lsel_ref,
    esel_ref,
    w1x_ref,
    w1s_ref,
    w1d_ref,
    w1r_ref,
    b1_ref,
    w2_ref,
    b2_ref,
    s2f_ref,
    s2fb_ref,
    w1cat_ref,
    b1cat_ref,
    w2cat_ref,
    cb2_ref,
    ginp_ref,
    locgin_ref,
    blg_ref,
    wihT_ref,
    whhT_ref,
    bih_ref,
    bhh_ref,
    cls_w_ref,
    cls_b_ref,
    logits_ref,
    gates_ref,
    state_ref,
    psum_ref,
    gsum_ref,
    *, n_steps):
    t = pl.program_id(1)

    @pl.when(t == 0)
    def _init():
        state_ref[...] = jnp.zeros_like(state_ref)
        psum_ref[...] = jnp.zeros_like(psum_ref)
        gsum_ref[...] = jnp.zeros_like(gsum_ref)

    wr = wr_ref[0, 0]
    wi = wi_ref[0, 0]
    p = p_ref[...]
    dmt = dmt_ref[...]
    lsel = lsel_ref[...]
    esel = esel_ref[...]

    def expand(x):
        return jnp.dot(x, p, preferred_element_type=jnp.float32)

    def rows_to_full(rows):
        return jnp.dot(esel, rows, preferred_element_type=jnp.float32)

    def src_full(x_exp):
        rows = jnp.dot(lsel, x_exp * dmt, preferred_element_type=jnp.float32)
        return rows_to_full(rows)

    wr_d = expand(wr)
    wi_d = expand(wi)
    wr_s = rows_to_full(wrrow_ref[0, 0])
    wi_s = rows_to_full(wirow_ref[0, 0])

    xr = wr_s * wr_d + wi_s * wi_d
    xi = wi_s * wr_d - wr_s * wi_d
    xmag = jnp.sqrt(xr * xr + xi * xi + 1e-12)
    xlog = jnp.log1p(xmag)
    gate = jnp.where((xi > 0.0) & ((xr <= 0.0) | (xi > xr)), 1.0, 0.0)
    gsum_ref[...] += jnp.sum(gate, keepdims=True)

    state = state_ref[...]
    sf = jnp.dot(state, s2f_ref[...],
                 preferred_element_type=jnp.float32) + s2fb_ref[...]
    sf_m = [sf[:, j * F:(j + 1) * F] for j in range(M)]
    rawt = rawt_ref[0, 0]

    w1s = w1s_ref[...]
    w1d = w1d_ref[...]
    w1r = w1r_ref[...]
    b1 = b1_ref[...]
    w1x = w1x_ref[...]
    w2 = w2_ref[...]
    b2 = b2_ref[...]

    a_s = [None] * M
    a_d = [None] * M
    for m in range(M):
        asm = sf_m[0] * w1s[0, m]
        adm = sf_m[0] * w1d[0, m]
        for j in range(1, M):
            asm = asm + sf_m[j] * w1s[j, m]
            adm = adm + sf_m[j] * w1d[j, m]
        a_s[m] = asm + rawt * w1r[0, m]
        a_d[m] = adm + rawt * w1r[1, m] + b1[0, m]

    h1 = [None] * M
    for m in range(M):
        pre = xlog * w1x[0, m] + src_full(expand(a_s[m])) + expand(a_d[m])
        h1[m] = jnp.maximum(pre, 0.0)

    pt = p.T
    agg = [None] * M
    for mo in range(M):
        acc = h1[0] * w2[0, mo]
        for mi in range(1, M):
            acc = acc + h1[mi] * w2[mi, mo]
        acc = (acc + b2[0, mo]) * gate
        agg[mo] = jnp.dot(acc, pt, preferred_element_type=jnp.float32)

    agg_cat = jnp.concatenate(agg, axis=1)

    win = rawwin_ref[0, 0]
    e1 = jnp.maximum(
        jnp.dot(win, w1cat_ref[...], preferred_element_type=jnp.float32)
        + b1cat_ref[...], 0.0) * mask_ref[0]
    enc_t = jnp.maximum(
        jnp.dot(e1, w2cat_ref[...], preferred_element_type=jnp.float32)
        + cb2_ref[...], 0.0)

    gru_in = (jnp.dot(agg_cat, ginp_ref[...], preferred_element_type=jnp.float32)
              + jnp.dot(enc_t, locgin_ref[...], preferred_element_type=jnp.float32)
              + blg_ref[...])

    gx = jnp.dot(gru_in, wihT_ref[...], preferred_element_type=jnp.float32) + bih_ref[...]
    gh = jnp.dot(state, whhT_ref[...], preferred_element_type=jnp.float32) + bhh_ref[...]
    xr_, xz_, xn_ = gx[:, :H], gx[:, H:2 * H], gx[:, 2 * H:]
    hr_, hz_, hn_ = gh[:, :H], gh[:, H:2 * H], gh[:, 2 * H:]
    r = jax.nn.sigmoid(xr_ + hr_)
    z = jax.nn.sigmoid(xz_ + hz_)
    n = jnp.tanh(xn_ + r * hn_)
    new_state = (1.0 - z) * n + z * state
    state_ref[...] = new_state

    pooled = jnp.dot(lsel, new_state,
                     preferred_element_type=jnp.float32) * (1.0 / C)
    psum_ref[...] += pooled

    @pl.when(t == n_steps - 1)
    def _finish():
        prev_mean = psum_ref[...] * (1.0 / n_steps)
        readout = jnp.concatenate([pooled, prev_mean], axis=1)
        logits_ref[0] = (
            jnp.dot(readout, cls_w_ref[...], preferred_element_type=jnp.float32)
            + cls_b_ref[...])
        gates_ref[0] = gsum_ref[...]


def kernel(raw_x, w_real, w_imag, conv1_w, conv1_b, conv2_w, conv2_b,
           msg_w1, msg_b1, msg_w2, msg_b2, loc_w, loc_b, s2f_w, s2f_b,
           gin_w, gin_b, gru_wih, gru_whh, gru_bih, gru_bhh, cls_w, cls_b):
    b, c, t_len = raw_x.shape
    n_steps = (t_len + STRIDE - 1) // STRIDE
    bg = b // G
    ts = np.arange(0, t_len, STRIDE)

    def group_stack(x):
        bb, nn = x.shape[0], x.shape[1]
        rest = x.shape[2:]
        y = x.reshape(bg, G, nn, *rest)
        y = jnp.moveaxis(y, 1, 2)
        return y.reshape(bg, nn, G * rest[0], *rest[1:])

    wr_t = jnp.transpose(w_real[:, :, ::STRIDE, :], (0, 2, 1, 3))
    wi_t = jnp.transpose(w_imag[:, :, ::STRIDE, :], (0, 2, 1, 3))
    wr_s = group_stack(wr_t)
    wi_s = group_stack(wi_t)
    wr_row = wr_t.reshape(bg, G, n_steps, CF).transpose(0, 2, 1, 3)
    wi_row = wi_t.reshape(bg, G, n_steps, CF).transpose(0, 2, 1, 3)
    raw_t = group_stack(
        jnp.transpose(raw_x[:, :, ::STRIDE], (0, 2, 1))[..., None])

    pad = (WIN - 1) // 2
    raw_pad = jnp.pad(raw_x, ((0, 0), (0, 0), (pad, pad)))
    win_idx = ts[:, None] + np.arange(WIN)[None, :]
    raw_win = group_stack(
        jnp.transpose(raw_pad[:, :, win_idx], (0, 2, 1, 3)))

    w1cat = jnp.zeros((WIN, K1 * ENC), dtype=jnp.float32)
    for j in range(K1):
        w1cat = w1cat.at[j:j + K1, j * ENC:(j + 1) * ENC].set(conv1_w[:, 0, :].T)
    b1cat = jnp.tile(conv1_b, (K1,)).reshape(1, K1 * ENC)
    w2cat = jnp.transpose(conv2_w, (2, 1, 0)).reshape(K1 * ENC, ENC)

    mask_np = np.ones((n_steps, 1, K1 * ENC), dtype=np.float32)
    for si, t0 in enumerate(ts):
        for j in range(K1):
            if not (0 <= t0 + j - 2 < t_len):
                mask_np[si, 0, j * ENC:(j + 1) * ENC] = 0.0
    mask = jnp.asarray(mask_np)

    p_np = np.zeros((F, CF), dtype=np.float32)
    for s in range(C):
        p_np[:, s * F:(s + 1) * F] = np.eye(F, dtype=np.float32)
    p_sel = jnp.asarray(p_np)

    d_np = np.zeros((C, CF), dtype=np.float32)
    for s in range(C):
        d_np[s, s * F:(s + 1) * F] = 1.0
    dmt = jnp.asarray(np.tile(d_np, (G, 1)))

    l_np = np.zeros((G, GC), dtype=np.float32)
    e_np = np.zeros((GC, G), dtype=np.float32)
    for g in range(G):
        l_np[g, g * C:(g + 1) * C] = 1.0
        e_np[g * C:(g + 1) * C, g] = 1.0
    lsel = jnp.asarray(l_np)
    esel = jnp.asarray(e_np)

    mi = 1 + 2 * M + 2
    w1x = msg_w1[0:1, :]
    w1s = msg_w1[1:1 + M, :]
    w1d = msg_w1[1 + M:1 + 2 * M, :]
    w1r = msg_w1[1 + 2 * M:mi, :]
    b1 = msg_b1.reshape(1, M)
    b2 = msg_b2.reshape(1, M)

    perm = np.arange(F * M).reshape(F, M).T.reshape(-1)
    s2fp = s2f_w[:, perm]
    s2fb = s2f_b[perm].reshape(1, F * M)
    ginp = gin_w[perm, :]
    locgin = loc_w @ gin_w
    blg = (loc_b @ gin_w + gin_b).reshape(1, H)
    wihT = gru_wih.T
    whhT = gru_whh.T
    bih = gru_bih.reshape(1, 3 * H)
    bhh = gru_bhh.reshape(1, 3 * H)
    cls_b2 = cls_b.reshape(1, N_CLASSES)

    def bspec(block, imap):
        return pl.BlockSpec(block, imap)

    grid = (bg, n_steps)
    full = lambda *shape: pl.BlockSpec(shape, lambda bi, ti: (0,) * len(shape))

    out_shapes = (
        jax.ShapeDtypeStruct((bg, G, N_CLASSES), jnp.float32),
        jax.ShapeDtypeStruct((bg, 1, 1), jnp.float32),
    )

    logits, gate_parts = pl.pallas_call(
        functools.partial(_step_kernel, n_steps=n_steps),
        grid=grid,
        in_specs=[
            bspec((1, 1, GC, F), lambda bi, ti: (bi, ti, 0, 0)),
            bspec((1, 1, GC, F), lambda bi, ti: (bi, ti, 0, 0)),
            bspec((1, 1, G, CF), lambda bi, ti: (bi, ti, 0, 0)),
            bspec((1, 1, G, CF), lambda bi, ti: (bi, ti, 0, 0)),
            bspec((1, 1, GC, 1), lambda bi, ti: (bi, ti, 0, 0)),
            bspec((1, 1, GC, WIN), lambda bi, ti: (bi, ti, 0, 0)),
            bspec((1, 1, K1 * ENC), lambda bi, ti: (ti, 0, 0)),
            full(F, CF),
            full(GC, CF),
            full(G, GC),
            full(GC, G),
            full(1, M),
            full(M, M),
            full(M, M),
            full(2, M),
            full(1, M),
            full(M, M),
            full(1, M),
            full(H, F * M),
            full(1, F * M),
            full(WIN, K1 * ENC),
            full(1, K1 * ENC),
            full(K1 * ENC, ENC),
            full(1, ENC),
            full(F * M, H),
            full(ENC, H),
            full(1, H),
            full(H, 3 * H),
            full(H, 3 * H),
            full(1, 3 * H),
            full(1, 3 * H),
            full(2 * H, N_CLASSES),
            full(1, N_CLASSES),
        ],
        out_specs=[
            bspec((1, G, N_CLASSES), lambda bi, ti: (bi, 0, 0)),
            bspec((1, 1, 1), lambda bi, ti: (bi, 0, 0)),
        ],
        out_shape=out_shapes,
        scratch_shapes=[
            pltpu.VMEM((GC, H), jnp.float32),
            pltpu.VMEM((G, H), jnp.float32),
            pltpu.VMEM((1, 1), jnp.float32),
        ],
        compiler_params=pltpu.CompilerParams(
            dimension_semantics=("parallel", "arbitrary"),
        ),
    )(
        wr_s, wi_s, wr_row, wi_row, raw_t, raw_win, mask,
        p_sel, dmt, lsel, esel,
        w1x, w1s, w1d, w1r, b1, msg_w2, b2,
        s2fp, s2fb, w1cat, b1cat, w2cat, conv2_b.reshape(1, ENC),
        ginp, locgin, blg, wihT, whhT, bih, bhh, cls_w, cls_b2,
    )

    e_num = C * (C - 1)
    gate_count = float(n_steps) * float(b * e_num * F)
    gate_rate = jnp.sum(gate_parts) / gate_count
    return logits.reshape(b, N_CLASSES), gate_rate


# speedup vs baseline: 39.5038x; 1.2174x over previous
import functools
import math

import jax
import jax.numpy as jnp
import numpy as np
from jax.experimental import pallas as pl
from jax.experimental.pallas import tpu as pltpu

C = 48
F = 16
M = 3
H = 32
ENC = 16
STRIDE = 10
THETA = math.radians(45.0)
N_CLASSES = 4
K1 = 5
WIN = 9
CF = C * F
G = 8
GC = G * C


def _step_kernel(
    wr_ref,
    wi_ref,
    wrrow_ref,
    wirow_ref,
    rawt_ref,
    rawwin_ref,
    mask_ref,
    p_ref,
    dmt_ref,
    lsel_ref,
    esel_ref,
    w1x_ref,
    w1s_ref,
    w1d_ref,
    w1r_ref,
    b1_ref,
    w2_ref,
    b2_ref,
    s2f_ref,
    s2fb_ref,
    w1cat_ref,
    b1cat_ref,
    w2cat_ref,
    cb2_ref,
    ginp_ref,
    locgin_ref,
    blg_ref,
    wihT_ref,
    whhT_ref,
    bih_ref,
    bhh_ref,
    cls_w_ref,
    cls_b_ref,
    logits_ref,
    gates_ref,
    state_ref,
    psum_ref,
    gsum_ref,
    *, n_steps):
    t = pl.program_id(1)

    @pl.when(t == 0)
    def _init():
        state_ref[...] = jnp.zeros_like(state_ref)
        psum_ref[...] = jnp.zeros_like(psum_ref)
        gsum_ref[...] = jnp.zeros_like(gsum_ref)

    wr = wr_ref[0, 0]
    wi = wi_ref[0, 0]
    p = p_ref[...]
    dmt = dmt_ref[...]
    lsel = lsel_ref[...]
    esel = esel_ref[...]

    def expand(x):
        return jnp.dot(x, p, preferred_element_type=jnp.float32)

    def rows_to_full(rows):
        return jnp.dot(esel, rows, preferred_element_type=jnp.float32)

    def src_full(x_exp):
        rows = jnp.dot(lsel, x_exp * dmt, preferred_element_type=jnp.float32)
        return rows_to_full(rows)

    wr_d = expand(wr)
    wi_d = expand(wi)
    wr_s = rows_to_full(wrrow_ref[0, 0])
    wi_s = rows_to_full(wirow_ref[0, 0])

    xr = wr_s * wr_d + wi_s * wi_d
    xi = wi_s * wr_d - wr_s * wi_d
    xmag = jnp.sqrt(xr * xr + xi * xi + 1e-12)
    xlog = jnp.log1p(xmag)
    gate = jnp.where((xi > 0.0) & ((xr <= 0.0) | (xi > xr)), 1.0, 0.0)
    gsum_ref[...] += jnp.sum(gate, keepdims=True)

    state = state_ref[...]
    sf = jnp.dot(state, s2f_ref[...],
                 preferred_element_type=jnp.float32) + s2fb_ref[...]
    sf_m = [sf[:, j * F:(j + 1) * F] for j in range(M)]
    rawt = rawt_ref[0, 0]

    w1s = w1s_ref[...]
    w1d = w1d_ref[...]
    w1r = w1r_ref[...]
    b1 = b1_ref[...]
    w1x = w1x_ref[...]
    w2 = w2_ref[...]
    b2 = b2_ref[...]

    a_s = [None] * M
    a_d = [None] * M
    for m in range(M):
        asm = sf_m[0] * w1s[0, m]
        adm = sf_m[0] * w1d[0, m]
        for j in range(1, M):
            asm = asm + sf_m[j] * w1s[j, m]
            adm = adm + sf_m[j] * w1d[j, m]
        a_s[m] = asm + rawt * w1r[0, m]
        a_d[m] = adm + rawt * w1r[1, m] + b1[0, m]

    h1 = [None] * M
    for m in range(M):
        pre = xlog * w1x[0, m] + src_full(expand(a_s[m])) + expand(a_d[m])
        h1[m] = jnp.maximum(pre, 0.0)

    pt = p.T
    agg = [None] * M
    for mo in range(M):
        acc = h1[0] * w2[0, mo]
        for mi in range(1, M):
            acc = acc + h1[mi] * w2[mi, mo]
        acc = (acc + b2[0, mo]) * gate
        agg[mo] = jnp.dot(acc, pt, preferred_element_type=jnp.float32)

    agg_cat = jnp.concatenate(agg, axis=1)

    win = rawwin_ref[0, 0]
    e1 = jnp.maximum(
        jnp.dot(win, w1cat_ref[...], preferred_element_type=jnp.float32)
        + b1cat_ref[...], 0.0) * mask_ref[0]
    enc_t = jnp.maximum(
        jnp.dot(e1, w2cat_ref[...], preferred_element_type=jnp.float32)
        + cb2_ref[...], 0.0)

    gru_in = (jnp.dot(agg_cat, ginp_ref[...], preferred_element_type=jnp.float32)
              + jnp.dot(enc_t, locgin_ref[...], preferred_element_type=jnp.float32)
              + blg_ref[...])

    gx = jnp.dot(gru_in, wihT_ref[...], preferred_element_type=jnp.float32) + bih_ref[...]
    gh = jnp.dot(state, whhT_ref[...], preferred_element_type=jnp.float32) + bhh_ref[...]
    xr_, xz_, xn_ = gx[:, :H], gx[:, H:2 * H], gx[:, 2 * H:]
    hr_, hz_, hn_ = gh[:, :H], gh[:, H:2 * H], gh[:, 2 * H:]
    r = jax.nn.sigmoid(xr_ + hr_)
    z = jax.nn.sigmoid(xz_ + hz_)
    n = jnp.tanh(xn_ + r * hn_)
    new_state = (1.0 - z) * n + z * state
    state_ref[...] = new_state

    pooled = jnp.dot(lsel, new_state,
                     preferred_element_type=jnp.float32) * (1.0 / C)
    psum_ref[...] += pooled

    @pl.when(t == n_steps - 1)
    def _finish():
        prev_mean = psum_ref[...] * (1.0 / n_steps)
        readout = jnp.concatenate([pooled, prev_mean], axis=1)
        logits_ref[0] = (
            jnp.dot(readout, cls_w_ref[...], preferred_element_type=jnp.float32)
            + cls_b_ref[...])
        gates_ref[0] = gsum_ref[...]


def kernel(raw_x, w_real, w_imag, conv1_w, conv1_b, conv2_w, conv2_b,
           msg_w1, msg_b1, msg_w2, msg_b2, loc_w, loc_b, s2f_w, s2f_b,
           gin_w, gin_b, gru_wih, gru_whh, gru_bih, gru_bhh, cls_w, cls_b):
    b, c, t_len = raw_x.shape
    n_steps = (t_len + STRIDE - 1) // STRIDE
    bg = b // G
    ts = np.arange(0, t_len, STRIDE)

    def group_stack(x):
        bb, nn = x.shape[0], x.shape[1]
        rest = x.shape[2:]
        y = x.reshape(bg, G, nn, *rest)
        y = jnp.moveaxis(y, 1, 2)
        return y.reshape(bg, nn, G * rest[0], *rest[1:])

    wr_t = jnp.transpose(w_real[:, :, ::STRIDE, :], (0, 2, 1, 3))
    wi_t = jnp.transpose(w_imag[:, :, ::STRIDE, :], (0, 2, 1, 3))
    wr_s = group_stack(wr_t)
    wi_s = group_stack(wi_t)
    wr_row = wr_t.reshape(bg, G, n_steps, CF).transpose(0, 2, 1, 3)
    wi_row = wi_t.reshape(bg, G, n_steps, CF).transpose(0, 2, 1, 3)
    raw_t = group_stack(
        jnp.transpose(raw_x[:, :, ::STRIDE], (0, 2, 1))[..., None])

    pad = (WIN - 1) // 2
    raw_pad = jnp.pad(raw_x, ((0, 0), (0, 0), (pad, pad)))
    win_idx = ts[:, None] + np.arange(WIN)[None, :]
    raw_win = group_stack(
        jnp.transpose(raw_pad[:, :, win_idx], (0, 2, 1, 3)))

    w1cat = jnp.zeros((WIN, K1 * ENC), dtype=jnp.float32)
    for j in range(K1):
        w1cat = w1cat.at[j:j + K1, j * ENC:(j + 1) * ENC].set(conv1_w[:, 0, :].T)
    b1cat = jnp.tile(conv1_b, (K1,)).reshape(1, K1 * ENC)
    w2cat = jnp.transpose(conv2_w, (2, 1, 0)).reshape(K1 * ENC, ENC)

    mask_np = np.ones((n_steps, 1, K1 * ENC), dtype=np.float32)
    for si, t0 in enumerate(ts):
        for j in range(K1):
            if not (0 <= t0 + j - 2 < t_len):
                mask_np[si, 0, j * ENC:(j + 1) * ENC] = 0.0
    mask = jnp.asarray(mask_np)

    p_np = np.zeros((F, CF), dtype=np.float32)
    for s in range(C):
        p_np[:, s * F:(s + 1) * F] = np.eye(F, dtype=np.float32)
    p_sel = jnp.asarray(p_np)

    d_np = np.zeros((C, CF), dtype=np.float32)
    for s in range(C):
        d_np[s, s * F:(s + 1) * F] = 1.0
    dmt = jnp.asarray(np.tile(d_np, (G, 1)))

    l_np = np.zeros((G, GC), dtype=np.float32)
    e_np = np.zeros((GC, G), dtype=np.float32)
    for g in range(G):
        l_np[g, g * C:(g + 1) * C] = 1.0
        e_np[g * C:(g + 1) * C, g] = 1.0
    lsel = jnp.asarray(l_np)
    esel = jnp.asarray(e_np)

    mi = 1 + 2 * M + 2
    w1x = msg_w1[0:1, :]
    w1s = msg_w1[1:1 + M, :]
    w1d = msg_w1[1 + M:1 + 2 * M, :]
    w1r = msg_w1[1 + 2 * M:mi, :]
    b1 = msg_b1.reshape(1, M)
    b2 = msg_b2.reshape(1, M)

    perm = np.arange(F * M).reshape(F, M).T.reshape(-1)
    s2fp = s2f_w[:, perm]
    s2fb = s2f_b[perm].reshape(1, F * M)
    ginp = gin_w[perm, :]
    locgin = loc_w @ gin_w
    blg = (loc_b @ gin_w + gin_b).reshape(1, H)
    wihT = gru_wih.T
    whhT = gru_whh.T
    bih = gru_bih.reshape(1, 3 * H)
    bhh = gru_bhh.reshape(1, 3 * H)
    cls_b2 = cls_b.reshape(1, N_CLASSES)

    def bspec(block, imap):
        return pl.BlockSpec(block, imap)

    grid = (bg, n_steps)
    full = lambda *shape: pl.BlockSpec(shape, lambda bi, ti: (0,) * len(shape))

    out_shapes = (
        jax.ShapeDtypeStruct((bg, G, N_CLASSES), jnp.float32),
        jax.ShapeDtypeStruct((bg, 1, 1), jnp.float32),
    )

    logits, gate_parts = pl.pallas_call(
        functools.partial(_step_kernel, n_steps=n_steps),
        grid=grid,
        in_specs=[
            bspec((1, 1, GC, F), lambda bi, ti: (bi, ti, 0, 0)),
            bspec((1, 1, GC, F), lambda bi, ti: (bi, ti, 0, 0)),
            bspec((1, 1, G, CF), lambda bi, ti: (bi, ti, 0, 0)),
            bspec((1, 1, G, CF), lambda bi, ti: (bi, ti, 0, 0)),
            bspec((1, 1, GC, 1), lambda bi, ti: (bi, ti, 0, 0)),
            bspec((1, 1, GC, WIN), lambda bi, ti: (bi, ti, 0, 0)),
            bspec((1, 1, K1 * ENC), lambda bi, ti: (ti, 0, 0)),
            full(F, CF),
            full(GC, CF),
            full(G, GC),
            full(GC, G),
            full(1, M),
            full(M, M),
            full(M, M),
            full(2, M),
            full(1, M),
            full(M, M),
            full(1, M),
            full(H, F * M),
            full(1, F * M),
            full(WIN, K1 * ENC),
            full(1, K1 * ENC),
            full(K1 * ENC, ENC),
            full(1, ENC),
            full(F * M, H),
            full(ENC, H),
            full(1, H),
            full(H, 3 * H),
            full(H, 3 * H),
            full(1, 3 * H),
            full(1, 3 * H),
            full(2 * H, N_CLASSES),
            full(1, N_CLASSES),
        ],
        out_specs=[
            bspec((1, G, N_CLASSES), lambda bi, ti: (bi, 0, 0)),
            bspec((1, 1, 1), lambda bi, ti: (bi, 0, 0)),
        ],
        out_shape=out_shapes,
        scratch_shapes=[
            pltpu.VMEM((GC, H), jnp.float32),
            pltpu.VMEM((G, H), jnp.float32),
            pltpu.VMEM((1, 1), jnp.float32),
        ],
        compiler_params=pltpu.CompilerParams(
            dimension_semantics=("parallel", "arbitrary"),
        ),
    )(
        wr_s, wi_s, wr_row, wi_row, raw_t, raw_win, mask,
        p_sel, dmt, lsel, esel,
        w1x, w1s, w1d, w1r, b1, msg_w2, b2,
        s2fp, s2fb, w1cat, b1cat, w2cat, conv2_b.reshape(1, ENC),
        ginp, locgin, blg, wihT, whhT, bih, bhh, cls_w, cls_b2,
    )

    e_num = C * (C - 1)
    gate_count = float(n_steps) * float(b * e_num * F)
    gate_rate = jnp.sum(gate_parts) / gate_count
    return logits.reshape(b, N_CLASSES), gate_rate
